# Initial kernel scaffold; baseline (speedup 1.0000x reference)
import jax
import jax.numpy as jnp
from jax.experimental import pallas as pl


def kernel(inputs, token_table, position_table):
    raise NotImplementedError("write your pallas kernel here")


# speedup vs baseline: 2.6391x; 2.6391x over previous
import functools

import jax
import jax.numpy as jnp
from jax import lax
from jax.experimental import pallas as pl
from jax.experimental.pallas import tpu as pltpu
from jax.experimental.pallas import tpu_sc as plsc

VOCAB = 100000
SEQ_LEN = 200
EMBED_DIM = 128
BATCH = 1024

NUM_CORES = 2
NUM_SUBCORES = 16
NUM_WORKERS = NUM_CORES * NUM_SUBCORES
BPW = BATCH // NUM_WORKERS
IDX_PER_W = BPW * SEQ_LEN
HALF = SEQ_LEN // 2
LANES = 16
GROUPS = EMBED_DIM // LANES
SCALE = float(EMBED_DIM) ** 0.5


def _embed_kernel(idx_hbm, table_hbm, pos_hbm, out_hbm, idx_v, pos_v, buf_v,
                  sem_in, sem_out):
    wid = lax.axis_index("s") * NUM_CORES + lax.axis_index("c")
    b0 = wid * BPW

    pltpu.sync_copy(idx_hbm.at[pl.ds(wid * IDX_PER_W, IDX_PER_W)], idx_v)
    pltpu.sync_copy(pos_hbm, pos_v)

    def rows16(base, mvec, lanes):
        for k in lanes:
            s = base + (k - lanes[0])
            mv = jnp.full((LANES,), mvec[k], jnp.float32)
            for g in range(GROUPS):
                sl = pl.ds(g * LANES, LANES)
                buf_v[s, sl] = (buf_v[s, sl] * SCALE + pos_v[s, sl]) * mv

    def batch_body(b, _):
        ib = b * SEQ_LEN
        cps = []
        for lo, n in ((0, 104), (104, 96)):
            cps.append(pltpu.async_copy(
                table_hbm.at[idx_v.at[pl.ds(ib + lo, n)]],
                buf_v.at[pl.ds(lo, n)],
                sem_in))
        for cp in cps:
            cp.wait()

        def grp_body(i, _):
            idxg = idx_v[pl.ds(ib + i * LANES, LANES)]
            mvec = jnp.where(idxg != 0, 1.0, 0.0).astype(jnp.float32)
            rows16(i * LANES, mvec, list(range(LANES)))
            return 0

        lax.fori_loop(0, SEQ_LEN // LANES, grp_body, 0)
        idxg = idx_v[pl.ds(ib + SEQ_LEN - LANES, LANES)]
        mvec = jnp.where(idxg != 0, 1.0, 0.0).astype(jnp.float32)
        rows16((SEQ_LEN // LANES) * LANES, mvec, list(range(8, LANES)))

        out_cp = pltpu.async_copy(buf_v, out_hbm.at[b0 + b], sem_out)
        out_cp.wait()
        return 0

    lax.fori_loop(0, BPW, batch_body, 0)


@jax.jit
def _embed(idx_flat, token_table, position_table):
    mesh = plsc.VectorSubcoreMesh(core_axis_name="c", subcore_axis_name="s")
    kern = functools.partial(
        pl.kernel,
        out_type=jax.ShapeDtypeStruct((BATCH, SEQ_LEN, EMBED_DIM),
                                      jnp.float32),
        mesh=mesh,
        scratch_types=[
            pltpu.VMEM((IDX_PER_W,), jnp.int32),
            pltpu.VMEM((SEQ_LEN, EMBED_DIM), jnp.float32),
            pltpu.VMEM((SEQ_LEN, EMBED_DIM), jnp.float32),
            pltpu.SemaphoreType.DMA,
            pltpu.SemaphoreType.DMA,
        ],
    )(_embed_kernel)
    return kern(idx_flat, token_table, position_table)


def kernel(inputs, token_table, position_table):
    idx_flat = inputs.astype(jnp.int32).reshape(-1)
    return _embed(idx_flat, token_table, position_table)
